# baseline (speedup 1.0000x reference)
import functools

import jax
import jax.numpy as jnp
from jax import lax
from jax.experimental import pallas as pl
from jax.experimental.pallas import tpu as pltpu
from jax.experimental.pallas import tpu_sc as plsc

D = 128
ROWS = 367
GMAX = 128
NBUF = 8
H0 = 128
H1 = 72


def kernel(time, pe):
    bsz, t = time.shape
    info = plsc.get_sparse_core_info()
    nc, ns = info.num_cores, info.num_subcores
    nw = nc * ns
    rpw = bsz // nw
    n_units = 2 * rpw
    n_outer = n_units // NBUF
    assert bsz == nw * rpw and n_units % NBUF == 0
    assert t == H0 + H1 and H0 <= GMAX and H1 <= GMAX

    mesh = plsc.VectorSubcoreMesh(core_axis_name="c", subcore_axis_name="s")

    @functools.partial(
        pl.kernel,
        mesh=mesh,
        compiler_params=pltpu.CompilerParams(use_tc_tiling_on_sc=True),
        out_type=jax.ShapeDtypeStruct((bsz, t, D), jnp.float32),
        scratch_types=[
            pltpu.VMEM_SHARED((ROWS, D), jnp.float32),
            pltpu.VMEM((rpw, t), jnp.int32),
        ]
        + [pltpu.VMEM((H0 if b % 2 == 0 else H1, D), jnp.float32)
           for b in range(NBUF)]
        + [pltpu.SemaphoreType.DMA for _ in range(2 * NBUF)],
    )
    def k(idx_hbm, pe_hbm, out_hbm, tab_sh, idx_v, *rest):
        rows = rest[:NBUF]
        gsem = rest[NBUF:2 * NBUF]
        ssem = rest[2 * NBUF:]

        c = lax.axis_index("c")
        s = lax.axis_index("s")
        wid = s * nc + c

        @pl.when(s == 0)
        def _():
            pltpu.sync_copy(pe_hbm, tab_sh)

        plsc.subcore_barrier()

        pltpu.sync_copy(idx_hbm.at[pl.ds(wid * rpw, rpw)], idx_v)

        def gather_unit(u, b, h):
            r = u // 2
            off, sz = (0, H0) if h == 0 else (H0, H1)
            pltpu.async_copy(
                tab_sh.at[idx_v.at[r, pl.ds(off, sz)]], rows[b], gsem[b])

        for b in range(NBUF):
            gather_unit(b, b, b % 2)

        def outer(o, carry):
            for b in range(NBUF):
                u = o * NBUF + b
                h = b % 2
                off, sz = (0, H0) if h == 0 else (H0, H1)
                pltpu.make_async_copy(
                    out_hbm.at[0].at[pl.ds(0, sz)], rows[b], gsem[b]).wait()
                pltpu.async_copy(
                    rows[b],
                    out_hbm.at[wid * rpw + u // 2].at[pl.ds(off, sz)],
                    ssem[b])
            for b in range(NBUF):
                h = b % 2
                sz = H0 if h == 0 else H1
                pltpu.make_async_copy(
                    rows[b], out_hbm.at[0].at[pl.ds(0, sz)], ssem[b]).wait()

                @pl.when(o < n_outer - 1)
                def _():
                    gather_unit((o + 1) * NBUF + b, b, h)
            return carry

        lax.fori_loop(0, n_outer, outer, 0)

    return k(time.astype(jnp.int32), pe)
